# baseline (speedup 1.0000x reference)
import functools

import jax
import jax.numpy as jnp
from jax import lax
from jax.experimental import pallas as pl
from jax.experimental.pallas import tpu as pltpu
from jax.experimental.pallas import tpu_sc as plsc

BATCH = 16384
D = 64
NC = 2
NS = 16
L = 16
NW = NC * NS
BPW = BATCH // NW
CHUNK = 128
NCHUNK = BPW // CHUNK


def _sc_body(uids, gids, utab, gtab, out, idx_u, idx_g, rows_u, rows_g,
             tbuf, out_v, sem):
    wid = lax.axis_index("s") * NC + lax.axis_index("c")
    base = wid * BPW

    for j in range(NCHUNK):
        pltpu.sync_copy(uids.at[pl.ds(base + j * CHUNK, CHUNK)], idx_u.at[j])
        pltpu.sync_copy(gids.at[pl.ds(base + j * CHUNK, CHUNK)], idx_g.at[j])

    copies = []
    for j in range(NCHUNK):
        copies.append(
            pltpu.async_copy(utab.at[idx_u.at[j]],
                             rows_u.at[pl.ds(j * CHUNK, CHUNK)], sem))
        copies.append(
            pltpu.async_copy(gtab.at[idx_g.at[j]],
                             rows_g.at[pl.ds(j * CHUNK, CHUNK)], sem))
    for c in copies:
        c.wait()

    lanes16 = lax.iota(jnp.int32, L) * L

    def group(g, carry):
        rbase = g * L
        for r in range(L):
            row = rbase + r
            acc = (rows_u[row, pl.ds(0, L)] * rows_g[row, pl.ds(0, L)])
            for c in range(1, D // L):
                acc = acc + (rows_u[row, pl.ds(c * L, L)] *
                             rows_g[row, pl.ds(c * L, L)])
            tbuf[pl.ds(r * L, L)] = acc
        tot = plsc.load_gather(tbuf, [lanes16])
        for c in range(1, L):
            tot = tot + plsc.load_gather(tbuf, [lanes16 + c])
        sig = 1.0 / (1.0 + jnp.exp(-tot))
        out_v[pl.ds(rbase, L)] = sig
        return carry

    lax.fori_loop(0, BPW // L, group, 0)

    pltpu.sync_copy(out_v, out.at[pl.ds(base, BPW)])


@jax.jit
def _two_tower(user_ids, game_ids, user_table, game_table):
    mesh = plsc.VectorSubcoreMesh(core_axis_name="c", subcore_axis_name="s")
    f = pl.kernel(
        _sc_body,
        out_type=jax.ShapeDtypeStruct((BATCH,), jnp.float32),
        mesh=mesh,
        scratch_types=[
            pltpu.VMEM((NCHUNK, CHUNK), jnp.int32),
            pltpu.VMEM((NCHUNK, CHUNK), jnp.int32),
            pltpu.VMEM((BPW, D), jnp.float32),
            pltpu.VMEM((BPW, D), jnp.float32),
            pltpu.VMEM((L * L,), jnp.float32),
            pltpu.VMEM((BPW,), jnp.float32),
            pltpu.SemaphoreType.DMA,
        ],
        compiler_params=pltpu.CompilerParams(
            needs_layout_passes=False, use_tc_tiling_on_sc=False),
    )
    return f(user_ids, game_ids, user_table, game_table)


def kernel(user_ids, game_ids, user_table, game_table):
    user_ids = user_ids.astype(jnp.int32)
    game_ids = game_ids.astype(jnp.int32)
    return _two_tower(user_ids, game_ids, user_table, game_table)
